# baseline (speedup 1.0000x reference)
import jax
import jax.numpy as jnp
from jax.experimental import pallas as pl
from jax.experimental.pallas import tpu as pltpu

_F32 = jnp.float32
_NB = 2048
_IMAX = (1 << 31) - 1


def _dot(a, b, precision=None):
    return jax.lax.dot_general(a, b, (((1,), (0,)), ((), ())),
                               preferred_element_type=_F32, precision=precision)


def _top3_weights_cols(d):
    iota = jax.lax.broadcasted_iota(jnp.int32, d.shape, 0) & 63
    di = (jax.lax.bitcast_convert_type(d, jnp.int32) & ~63) | iota
    for _ in range(3):
        m = jnp.min(di, axis=0, keepdims=True)
        di = jnp.where(di == m, _IMAX, di)
    sel = di == _IMAX
    s_sum = jnp.sum(jnp.where(sel, d, 0.0), axis=0, keepdims=True)
    return jnp.where(sel, 1.0 - d * (1.0 / s_sum), 0.0)


def _dist_cols(nodes_t, pts):
    d2 = None
    for c in range(3):
        diff = nodes_t[:, c:c + 1] - pts[c:c + 1, :]
        d2 = diff * diff if d2 is None else d2 + diff * diff
    return jnp.sqrt(d2)


def _fused_kernel(nbf_ref, naf_ref, g_ref, ig_ref, s16_ref, s32_ref,
                  na_cat_ref, nbt_cat_ref,
                  W1_ref, b1_ref, W2_ref, b2_ref,
                  V1_ref, c1_ref, V2_ref, c2_ref, V3_ref, c3_ref,
                  A1_ref, a1_ref, A2_ref, a2_ref,
                  U1_ref, u1_ref, U2_ref, u2_ref, U3_ref, u3_ref,
                  p_ref, ii_ref, f1_ref, f2_ref, nb_t_ref, na_ref,
                  P1_ref, p1_ref, P2_ref, p2_ref, P3_ref, p3_ref,
                  coarse_ref, fine_ref,
                  mb_s, ma_s):
    b = pl.program_id(0)
    i = pl.program_id(1)
    relu = jax.nn.relu
    Mb = nb_t_ref.shape[1]
    Ma = na_ref.shape[2]

    @pl.when(jnp.logical_and(b == 0, i == 0))
    def _stage1():
        nbf = nbf_ref[...]
        naf = naf_ref[...]
        g = g_ref[...]
        ig = ig_ref[...]
        nB = g.shape[1]
        wb = nbf.shape[1]
        wa = naf.shape[1]
        selb = (jax.lax.broadcasted_iota(jnp.int32, (nB, wb), 1) // Mb
                == jax.lax.broadcasted_iota(jnp.int32, (nB, wb), 0)
                ).astype(_F32)
        sela = (jax.lax.broadcasted_iota(jnp.int32, (nB, wa), 1) // Ma
                == jax.lax.broadcasted_iota(jnp.int32, (nB, wa), 0)
                ).astype(_F32)
        mcol_b = selb[1:2, :]
        mcol_a = sela[1:2, :]

        t = relu(_dot(W1_ref[:, :256], nbf)
                 + _dot(_dot(W1_ref[:, 256:], ig), selb) + b1_ref[...])
        nb_att = _dot(W2_ref[...], t) + b2_ref[...]
        nb_att_bd = jnp.concatenate(
            [nb_att * (1.0 - mcol_b), nb_att * mcol_b], axis=0)
        nb_w = _dot(s32_ref[...], nb_att_bd) * (1.0 / 80.0)
        q = relu(_dot(V1_ref[:, :256], nbf)
                 + _dot(_dot(V1_ref[:, 256:768], g), selb)
                 + _dot(V1_ref[:, 768:1280], nb_w)
                 + _dot(_dot(V1_ref[:, 1280:], ig), selb) + c1_ref[...])
        q = relu(_dot(V2_ref[...], q) + c2_ref[...])
        up_nb = _dot(V3_ref[...], q) + c3_ref[...]
        mb_cat = _dot(P1_ref[:, 128:640], up_nb)
        mb_s[0] = mb_cat[:, :Mb]
        mb_s[1] = mb_cat[:, Mb:]
        r = relu(_dot(A1_ref[:, :64], naf)
                 + _dot(_dot(A1_ref[:, 64:], ig), sela) + a1_ref[...])
        na_att = _dot(A2_ref[...], r) + a2_ref[...]
        na_att_bd = jnp.concatenate(
            [na_att * (1.0 - mcol_a), na_att * mcol_a], axis=0)
        na_w = _dot(s16_ref[...], na_att_bd) * (1.0 / 320.0)
        d = _dist_cols(nbt_cat_ref[...], na_cat_ref[...])
        row_b = jax.lax.broadcasted_iota(jnp.int32, d.shape, 0) // Mb
        col_b = jax.lax.broadcasted_iota(jnp.int32, d.shape, 1) // Ma
        d = jnp.where(row_b == col_b, d, 1e30)
        s_sel = _top3_weights_cols(d)
        interp_ab = _dot(up_nb, s_sel)
        z = relu(_dot(U1_ref[:, :64], naf)
                 + _dot(U1_ref[:, 64:576], interp_ab)
                 + _dot(U1_ref[:, 576:], na_w) + u1_ref[...])
        z = relu(_dot(U2_ref[...], z) + u2_ref[...])
        up_na = _dot(U3_ref[...], z) + u3_ref[...]
        ma_cat = _dot(P1_ref[:, :128], up_na)
        ma_s[0] = ma_cat[:, :Ma]
        ma_s[1] = ma_cat[:, Ma:]

    p = p_ref[0]
    na = na_ref[0]

    d = _dist_cols(nb_t_ref[0], p)
    s_sel = _top3_weights_cols(d)

    ii = ii_ref[0]
    iota = jax.lax.broadcasted_iota(jnp.int32, (256, p.shape[1]), 0)
    ohs, ds = [], []
    for k in range(3):
        oh = (iota == ii[k:k + 1, :]).astype(_F32)
        coords = _dot(na, oh)
        diff = p - coords
        ds.append(jnp.sqrt(jnp.sum(diff * diff, axis=0, keepdims=True)))
        ohs.append(oh)
    rs = 1.0 / (ds[0] + ds[1] + ds[2])
    s_a = (1.0 - ds[0] * rs) * ohs[0]
    for k in range(1, 3):
        s_a = s_a + (1.0 - ds[k] * rs) * ohs[k]

    h = relu(_dot(mb_s[b], s_sel) + _dot(ma_s[b], s_a)
             + _dot(P1_ref[:, 640:672], f1_ref[0])
             + _dot(P1_ref[:, 672:], f2_ref[0])
             + p1_ref[...])
    h = relu(_dot(P2_ref[...], h) + p2_ref[...])
    o = _dot(P3_ref[...], h) + p3_ref[...]
    coarse_ref[0] = o[0:2, :]
    fine_ref[0] = o[2:82, :]


def kernel(pc, node_a, node_b, first_pn_out, second_pn_out, node_a_features,
           node_b_features, global_feature, img_s16_feature_map,
           img_s32_feature_map, img_global_feature, params, node_a_min_k_idx):
    B, N = pc.shape[0], pc.shape[2]
    Ma, Mb = node_a.shape[2], node_b.shape[2]
    f32 = _F32

    wide = lambda x: x.transpose(1, 0, 2).reshape(x.shape[1], -1)
    s16 = wide(img_s16_feature_map.reshape(B, img_s16_feature_map.shape[1], -1))
    s32 = wide(img_s32_feature_map.reshape(B, img_s32_feature_map.shape[1], -1))
    ig = img_global_feature.reshape(B, img_global_feature.shape[1]).T
    g = global_feature.reshape(B, global_feature.shape[1]).T
    nbf = wide(node_b_features)
    naf = wide(node_a_features)
    na_cat = wide(node_a)
    nb_t = node_b.transpose(0, 2, 1)
    nbt_cat = nb_t.reshape(B * Mb, 3)
    ii_t = node_a_min_k_idx.astype(jnp.int32).transpose(0, 2, 1)

    col = lambda v: v.reshape(-1, 1)

    (W1, b1), (W2, b2) = params['nb_att']
    (V1, c1), (V2, c2), (V3, c3) = params['nb_pn']
    (A1, a1), (A2, a2) = params['na_att']
    (U1, u1), (U2, u2), (U3, u3) = params['na_pn']
    (P1, q1), (P2, q2), (P3, q3) = params['pp_pn']

    w_s1 = [W1, col(b1), W2, col(b2),
            V1, col(c1), V2, col(c2), V3, col(c3),
            A1, col(a1), A2, col(a2),
            U1, col(u1), U2, col(u2), U3, col(u3)]
    w_s2 = [P1, col(q1), P2, col(q2), P3, col(q3)]

    wspec = lambda w: pl.BlockSpec(w.shape, lambda b, i: (0,) * w.ndim)
    bspec = lambda *s: pl.BlockSpec((1,) + s, lambda b, i: (b, 0, 0))
    pspec = lambda *s: pl.BlockSpec((1,) + s, lambda b, i: (b, 0, i))

    coarse, fine = pl.pallas_call(
        _fused_kernel,
        grid=(B, N // _NB),
        in_specs=[wspec(nbf), wspec(naf), wspec(g), wspec(ig),
                  wspec(s16), wspec(s32), wspec(na_cat), wspec(nbt_cat)]
                 + [wspec(w) for w in w_s1]
                 + [pspec(3, _NB), pspec(3, _NB), pspec(32, _NB),
                    pspec(64, _NB), bspec(Mb, 3), bspec(3, Ma)]
                 + [wspec(w) for w in w_s2],
        out_specs=[pl.BlockSpec((1, 2, _NB), lambda b, i: (b, 0, i)),
                   pl.BlockSpec((1, 80, _NB), lambda b, i: (b, 0, i))],
        out_shape=[jax.ShapeDtypeStruct((B, 2, N), f32),
                   jax.ShapeDtypeStruct((B, 80, N), f32)],
        scratch_shapes=[pltpu.VMEM((B, 256, Mb), f32),
                        pltpu.VMEM((B, 256, Ma), f32)],
        compiler_params=pltpu.CompilerParams(
            dimension_semantics=("arbitrary", "arbitrary")),
    )(nbf, naf, g, ig, s16, s32, na_cat, nbt_cat, *w_s1,
      pc, ii_t, first_pn_out, second_pn_out, nb_t, node_a, *w_s2)

    return (coarse, fine)
